# baseline (speedup 1.0000x reference)
import jax
import jax.numpy as jnp
from jax.experimental import pallas as pl

_B = 64


def _emb_kernel(x_ref, w_ref, o_ref):
    xb = x_ref[...]
    nv = xb.shape[2]
    m = jnp.max(xb, axis=2, keepdims=True)
    iota = jax.lax.broadcasted_iota(jnp.int32, xb.shape, 2)
    idx = jnp.min(jnp.where(xb == m, iota, nv), axis=2, keepdims=True)
    onehot = (iota == idx).astype(jnp.float32)
    w = w_ref[...]
    for b in range(xb.shape[0]):
        o_ref[b] = jnp.dot(onehot[b], w,
                           preferred_element_type=jnp.float32)


def kernel(x, W):
    B, S, NV = x.shape
    E = W.shape[1]
    return pl.pallas_call(
        _emb_kernel,
        grid=(B // _B,),
        in_specs=[
            pl.BlockSpec((_B, S, NV), lambda i: (i, 0, 0)),
            pl.BlockSpec((NV, E), lambda i: (0, 0)),
        ],
        out_specs=pl.BlockSpec((_B, S, E), lambda i: (i, 0, 0)),
        out_shape=jax.ShapeDtypeStruct((B, S, E), jnp.float32),
    )(x, W)


# speedup vs baseline: 1.0017x; 1.0017x over previous
import jax
import jax.numpy as jnp
from jax.experimental import pallas as pl
from jax.experimental.pallas import tpu as pltpu

_B = 64


def _emb_kernel(x_ref, w_ref, o_ref):
    xb = x_ref[...]
    nv = xb.shape[2]
    m = jnp.max(xb, axis=2, keepdims=True)
    iota = jax.lax.broadcasted_iota(jnp.int32, xb.shape, 2)
    idx = jnp.min(jnp.where(xb == m, iota, nv), axis=2, keepdims=True)
    onehot = (iota == idx).astype(jnp.float32)
    w = w_ref[...]
    for b in range(xb.shape[0]):
        o_ref[b] = jnp.dot(onehot[b], w,
                           preferred_element_type=jnp.float32)


def kernel(x, W):
    B, S, NV = x.shape
    E = W.shape[1]
    return pl.pallas_call(
        _emb_kernel,
        grid=(B // _B,),
        in_specs=[
            pl.BlockSpec((_B, S, NV), lambda i: (i, 0, 0)),
            pl.BlockSpec((NV, E), lambda i: (0, 0)),
        ],
        out_specs=pl.BlockSpec((_B, S, E), lambda i: (i, 0, 0)),
        out_shape=jax.ShapeDtypeStruct((B, S, E), jnp.float32),
        compiler_params=pltpu.CompilerParams(dimension_semantics=("parallel",)),
    )(x, W)
